# Initial kernel scaffold; baseline (speedup 1.0000x reference)
import jax
import jax.numpy as jnp
from jax.experimental import pallas as pl


def kernel(attn, sims, sinds):
    raise NotImplementedError("write your pallas kernel here")


# speedup vs baseline: 3.5325x; 3.5325x over previous
import functools

import jax
import jax.numpy as jnp
from jax import lax
from jax.experimental import pallas as pl
from jax.experimental.pallas import tpu as pltpu

NSP_ = 9
EPS = 1e-10


def _row_kernel(c_ref, attn_ref, sims_ref, sinds_ref, out_ref):
    B, H, W, NSP = sims_ref.shape
    HD = attn_ref.shape[1]
    K = attn_ref.shape[-1]
    ws = 7
    off = ws // 2

    h = pl.program_id(1)
    c = c_ref[0, 0]

    iota_v = lax.broadcasted_iota(jnp.int32, (W, K), 1)

    ph_rows = []
    for i in range(ws):
        r = jnp.clip(h - off + i, 0, H - 1)
        sims_r = sims_ref[0, r]
        sinds_r = sinds_ref[0, r]
        ph = jnp.zeros((W, K), dtype=jnp.float32)
        for s in range(NSP):
            ph = ph + jnp.where(sinds_r[:, s][:, None] == iota_v,
                                sims_r[:, s][:, None], 0.0)
        ph_rows.append(ph)

    sid = sinds_ref[0, h]
    oh = [sid[:, s][:, None] == iota_v for s in range(NSP)]

    match_s = []
    for s in range(NSP):
        cols = []
        for dh in range(ws):
            ph = ph_rows[dh]
            ph_pad = jnp.concatenate(
                [jnp.broadcast_to(ph[0:1], (off, K)), ph,
                 jnp.broadcast_to(ph[W - 1:W], (off, K))], axis=0)
            for dw in range(ws):
                phs = ph_pad[dw:dw + W]
                mk = jnp.sum(jnp.where(oh[s], phs, 0.0), axis=1, keepdims=True)
                cols.append(mk)
        match_s.append(jnp.concatenate(cols, axis=1))

    e = jnp.exp(attn_ref[0, :, 0] - c)
    for hd in range(HD):
        for s in range(NSP):
            em = e[hd] * match_s[s]
            den = jnp.sum(em, axis=1, keepdims=True)
            out_ref[0, hd, s, 0] = em / (EPS + den)


@jax.jit
def kernel(attn, sims, sinds):
    B, HD, H, W, K = attn.shape
    NSP = sims.shape[-1]
    c = jnp.max(attn).reshape(1, 1)

    grid = (B, H)
    out = pl.pallas_call(
        _row_kernel,
        grid=grid,
        in_specs=[
            pl.BlockSpec((1, 1), lambda b, h: (0, 0)),
            pl.BlockSpec((1, HD, 1, W, K), lambda b, h: (b, 0, h, 0, 0)),
            pl.BlockSpec((1, H, W, NSP), lambda b, h: (b, 0, 0, 0)),
            pl.BlockSpec((1, H, W, NSP), lambda b, h: (b, 0, 0, 0)),
        ],
        out_specs=pl.BlockSpec((1, HD, NSP, 1, W, K),
                               lambda b, h: (b, 0, 0, h, 0, 0)),
        out_shape=jax.ShapeDtypeStruct((B, HD, NSP, H, W, K), jnp.float32),
        compiler_params=pltpu.CompilerParams(
            dimension_semantics=("parallel", "parallel")),
    )(c, attn, sims, sinds)
    return out


# speedup vs baseline: 4.7945x; 1.3573x over previous
import jax
import jax.numpy as jnp
from jax import lax
from jax.experimental import pallas as pl
from jax.experimental.pallas import tpu as pltpu
from jax.experimental.pallas import tpu_sc as plsc

NSP = 9
EPS = 1e-10
WS = 7
OFF = WS // 2
HALO = 10
LANES = 16
NUM_TEC = 16
H_, W_, K_ = 56, 56, 49
WK = W_ * K_
WKP = WK + 8
PIXROW = W_ * NSP
NCHUNK = (WK + LANES - 1) // LANES


def _sc_match_body(sims_hbm, sinds_hbm, tbl_hbm, out_hbm,
                   sims_v, sinds_v, ph_v, tbl_v, mb_v):
    wid = lax.axis_index("c") * NUM_TEC + lax.axis_index("s")
    b = wid // 16
    wi = wid - b * 16
    nr = jnp.where(wi < 8, 4, 3)
    r0 = jnp.where(wi < 8, 4 * wi, 32 + 3 * (wi - 8))
    lo = jnp.clip(r0 - OFF, 0, H_ - HALO)

    inoff = pl.multiple_of((b * H_ + lo) * PIXROW, 8)
    pltpu.sync_copy(tbl_hbm, tbl_v)
    pltpu.sync_copy(sims_hbm.at[pl.ds(inoff, HALO * PIXROW)], sims_v)
    pltpu.sync_copy(sinds_hbm.at[pl.ds(inoff, HALO * PIXROW)], sinds_v)

    zero = jnp.zeros((LANES,), jnp.float32)

    def zbody(i, _):
        ph_v[pl.ds(i * LANES, LANES)] = zero
        return 0
    lax.fori_loop(0, (HALO * WK) // LANES, zbody, 0)

    iota = lax.broadcasted_iota(jnp.int32, (LANES,), 0)
    pix9 = iota * NSP
    i49 = iota * K_

    def sbody(ci, _):
        for sp in range(NSP):
            idxv = pix9 + (ci * (LANES * NSP) + sp)
            sindv = plsc.load_gather(sinds_v, [idxv])
            simsv = plsc.load_gather(sims_v, [idxv])
            pidx = i49 + ci * (LANES * K_) + sindv
            plsc.addupdate_scatter(ph_v, [pidx], simsv)
        return 0
    lax.fori_loop(0, (HALO * W_) // LANES, sbody, 0)

    def row_body(r, _):
        h = r0 + r
        rl9 = (h - lo) * PIXROW

        def cbody(c, _):
            base = c * LANES
            dhv = tbl_v[pl.ds(base, LANES)]
            col49v = tbl_v[pl.ds(WKP + base, LANES)]
            w9v = tbl_v[pl.ds(2 * WKP + base, LANES)]
            rv = jnp.clip(dhv + h, 0, H_ - 1)
            geo = (rv - lo) * WK + col49v
            for sp in range(NSP):
                sidv = plsc.load_gather(sinds_v, [w9v + (rl9 + sp)])
                val = plsc.load_gather(ph_v, [geo + sidv])
                mb_v[pl.ds(sp * WKP + base, LANES)] = val
            return 0
        lax.fori_loop(0, NCHUNK, cbody, 0)
        rowoff = (b * H_ + h) * (NSP * WK)
        for sp in range(NSP):
            pltpu.sync_copy(mb_v.at[pl.ds(sp * WKP, WK)],
                            out_hbm.at[pl.ds(pl.multiple_of(
                                rowoff + sp * WK, 8), WK)])
        return 0
    lax.fori_loop(0, nr, row_body, 0)


def _dense_kernel(c_ref, attn_ref, match_ref, sel1_ref, sel2_ref, out_ref):
    HD = attn_ref.shape[1]
    R = attn_ref.shape[2]
    c = c_ref[0, 0]
    for rr in range(R):
        e = jnp.exp(attn_ref[0, :, rr, :] - c)
        m = match_ref[0, rr]
        em = (e[:, None, :] * m[None, :, :]).reshape(HD * NSP, WK)
        den = lax.dot_general(em, sel1_ref[:, :], (((1,), (0,)), ((), ())),
                              preferred_element_type=jnp.float32)
        r = 1.0 / (EPS + den)
        rbc = lax.dot_general(r, sel2_ref[:, :], (((1,), (0,)), ((), ())),
                              preferred_element_type=jnp.float32)
        out_ref[0, :, :, rr, :] = (em * rbc).reshape(HD, NSP, WK)


@jax.jit
def kernel(attn, sims, sinds):
    B, HD, H, W, K = attn.shape
    c = jnp.max(attn).reshape(1, 1)

    f = jnp.arange(WK, dtype=jnp.int32)
    wcol = f // K
    kk = f - wcol * K
    dh = kk // WS - OFF
    dw = kk - (kk // WS) * WS - OFF
    col49 = jnp.clip(wcol + dw, 0, W - 1) * K
    pad = jnp.zeros((WKP - WK,), jnp.int32)
    tbl = jnp.concatenate([dh, pad, col49, pad, wcol * NSP, pad]).astype(jnp.int32)

    sims_flat = sims.reshape(B * H * W * NSP)
    sinds_flat = sinds.reshape(B * H * W * NSP)

    mesh = plsc.VectorSubcoreMesh(core_axis_name="c", subcore_axis_name="s")
    match_flat = pl.kernel(
        _sc_match_body,
        out_type=jax.ShapeDtypeStruct((B * H * NSP * WK,), jnp.float32),
        mesh=mesh,
        compiler_params=pltpu.CompilerParams(needs_layout_passes=False),
        scratch_types=[
            pltpu.VMEM((HALO * W * NSP,), jnp.float32),
            pltpu.VMEM((HALO * W * NSP,), jnp.int32),
            pltpu.VMEM((HALO * WK,), jnp.float32),
            pltpu.VMEM((3 * WKP,), jnp.int32),
            pltpu.VMEM((NSP * WKP,), jnp.float32),
        ],
    )(sims_flat, sinds_flat, tbl)

    sel1 = (wcol[:, None] == jnp.arange(W, dtype=jnp.int32)[None, :]
            ).astype(jnp.float32)
    sel2 = sel1.T

    attn_flat = attn.reshape(B, HD, H, WK)
    match5 = match_flat.reshape(B, H, NSP, WK)

    RB = 8
    out_flat = pl.pallas_call(
        _dense_kernel,
        grid=(B, H // RB),
        in_specs=[
            pl.BlockSpec((1, 1), lambda b, h: (0, 0)),
            pl.BlockSpec((1, HD, RB, WK), lambda b, h: (b, 0, h, 0)),
            pl.BlockSpec((1, RB, NSP, WK), lambda b, h: (b, h, 0, 0)),
            pl.BlockSpec((WK, W), lambda b, h: (0, 0)),
            pl.BlockSpec((W, WK), lambda b, h: (0, 0)),
        ],
        out_specs=pl.BlockSpec((1, HD, NSP, RB, WK),
                               lambda b, h: (b, 0, 0, h, 0)),
        out_shape=jax.ShapeDtypeStruct((B, HD, NSP, H, WK), jnp.float32),
        compiler_params=pltpu.CompilerParams(
            dimension_semantics=("parallel", "parallel")),
    )(c, attn_flat, match5, sel1, sel2)
    return out_flat.reshape(B, HD, NSP, H, W, K)


# speedup vs baseline: 7.0660x; 1.4738x over previous
import jax
import jax.numpy as jnp
from jax import lax
from jax.experimental import pallas as pl
from jax.experimental.pallas import tpu as pltpu
from jax.experimental.pallas import tpu_sc as plsc

NSP = 9
EPS = 1e-10
WS = 7
OFF = WS // 2
HALO = 10
LANES = 16
NUM_TEC = 16
H_, W_, K_ = 56, 56, 49
KP = 128
WK = W_ * K_
WKP = WK + 8
WKPAD = W_ * KP
ROWPAD = NSP * WKPAD
PIXROW = W_ * NSP
NCHUNK = (WK + LANES - 1) // LANES


def _sc_match_body(sims_hbm, sinds_hbm, tbl_hbm, out_hbm,
                   sims_v, sinds_v, ph_v, tbl_v, mb_v):
    wid = lax.axis_index("c") * NUM_TEC + lax.axis_index("s")
    b = wid // 16
    wi = wid - b * 16
    nr = jnp.where(wi < 8, 4, 3)
    r0 = jnp.where(wi < 8, 4 * wi, 32 + 3 * (wi - 8))
    lo = jnp.clip(r0 - OFF, 0, H_ - HALO)

    inoff = pl.multiple_of((b * H_ + lo) * PIXROW, 8)
    pltpu.sync_copy(tbl_hbm, tbl_v)
    pltpu.sync_copy(sims_hbm.at[pl.ds(inoff, HALO * PIXROW)], sims_v)
    pltpu.sync_copy(sinds_hbm.at[pl.ds(inoff, HALO * PIXROW)], sinds_v)

    zero = jnp.zeros((LANES,), jnp.float32)

    def zbody(i, _):
        ph_v[pl.ds(i * LANES, LANES)] = zero
        return 0
    lax.fori_loop(0, (HALO * WK) // LANES, zbody, 0)

    iota = lax.broadcasted_iota(jnp.int32, (LANES,), 0)
    pix9 = iota * NSP
    i49 = iota * K_

    def sbody(ci, _):
        for sp in range(NSP):
            idxv = pix9 + (ci * (LANES * NSP) + sp)
            sindv = plsc.load_gather(sinds_v, [idxv])
            simsv = plsc.load_gather(sims_v, [idxv])
            pidx = i49 + ci * (LANES * K_) + sindv
            plsc.addupdate_scatter(ph_v, [pidx], simsv)
        return 0
    lax.fori_loop(0, (HALO * W_) // LANES, sbody, 0)

    def row_body(r, _):
        h = r0 + r
        rl9 = (h - lo) * PIXROW

        def cbody(c, _):
            base = c * LANES
            dhv = tbl_v[pl.ds(base, LANES)]
            col49v = tbl_v[pl.ds(WKP + base, LANES)]
            w9v = tbl_v[pl.ds(2 * WKP + base, LANES)]
            didxv = tbl_v[pl.ds(3 * WKP + base, LANES)]
            rv = jnp.clip(dhv + h, 0, H_ - 1)
            geo = (rv - lo) * WK + col49v
            for sp in range(NSP):
                sidv = plsc.load_gather(sinds_v, [w9v + (rl9 + sp)])
                val = plsc.load_gather(ph_v, [geo + sidv])
                plsc.store_scatter(mb_v, [didxv + sp * WKPAD], val)
            return 0
        lax.fori_loop(0, NCHUNK, cbody, 0)
        rowoff = (b * H_ + h) * ROWPAD
        pltpu.sync_copy(mb_v, out_hbm.at[pl.ds(pl.multiple_of(rowoff, 8),
                                               ROWPAD)])
        return 0
    lax.fori_loop(0, nr, row_body, 0)


def _dense_kernel(c_ref, attn_ref, match_ref, out_ref):
    HD = attn_ref.shape[1]
    R = attn_ref.shape[2]
    c = c_ref[0, 0]
    for rr in range(R):
        ms = [match_ref[0, rr, s][:, 0:K_] for s in range(NSP)]
        for hd in range(HD):
            e = jnp.exp(attn_ref[0, hd, rr] - c)
            for s in range(NSP):
                em = e * ms[s]
                den = jnp.sum(em, axis=1, keepdims=True)
                out_ref[0, hd, s, rr] = em * (1.0 / (EPS + den))


@jax.jit
def kernel(attn, sims, sinds):
    B, HD, H, W, K = attn.shape
    c = jnp.max(attn).reshape(1, 1)

    f = jnp.arange(WK, dtype=jnp.int32)
    wcol = f // K
    kk = f - wcol * K
    dh = kk // WS - OFF
    dw = kk - (kk // WS) * WS - OFF
    col49 = jnp.clip(wcol + dw, 0, W - 1) * K
    didx = wcol * KP + kk
    pad = jnp.zeros((WKP - WK,), jnp.int32)
    dpad = jnp.full((WKP - WK,), KP - 1, jnp.int32)
    tbl = jnp.concatenate([dh, pad, col49, pad, wcol * NSP, pad,
                           didx, dpad]).astype(jnp.int32)

    sims_flat = sims.reshape(B * H * W * NSP)
    sinds_flat = sinds.reshape(B * H * W * NSP)

    mesh = plsc.VectorSubcoreMesh(core_axis_name="c", subcore_axis_name="s")
    match_flat = pl.kernel(
        _sc_match_body,
        out_type=jax.ShapeDtypeStruct((B * H * ROWPAD,), jnp.float32),
        mesh=mesh,
        compiler_params=pltpu.CompilerParams(needs_layout_passes=False),
        scratch_types=[
            pltpu.VMEM((HALO * W * NSP,), jnp.float32),
            pltpu.VMEM((HALO * W * NSP,), jnp.int32),
            pltpu.VMEM((HALO * WK,), jnp.float32),
            pltpu.VMEM((4 * WKP,), jnp.int32),
            pltpu.VMEM((ROWPAD,), jnp.float32),
        ],
    )(sims_flat, sinds_flat, tbl)

    match5 = match_flat.reshape(B, H, NSP, W, KP)

    RB = 8
    out = pl.pallas_call(
        _dense_kernel,
        grid=(B, H // RB),
        in_specs=[
            pl.BlockSpec((1, 1), lambda b, h: (0, 0)),
            pl.BlockSpec((1, HD, RB, W, K), lambda b, h: (b, 0, h, 0, 0)),
            pl.BlockSpec((1, RB, NSP, W, KP), lambda b, h: (b, h, 0, 0, 0)),
        ],
        out_specs=pl.BlockSpec((1, HD, NSP, RB, W, K),
                               lambda b, h: (b, 0, 0, h, 0, 0)),
        out_shape=jax.ShapeDtypeStruct((B, HD, NSP, H, W, K), jnp.float32),
        compiler_params=pltpu.CompilerParams(
            dimension_semantics=("parallel", "parallel")),
    )(c, attn, match5)
    return out


# speedup vs baseline: 10.5153x; 1.4882x over previous
import functools

import jax
import jax.numpy as jnp
from jax import lax
from jax.experimental import pallas as pl
from jax.experimental.pallas import tpu as pltpu
from jax.experimental.pallas import tpu_sc as plsc

NSP = 9
EPS = 1e-10
WS = 7
OFF = WS // 2
LANES = 16
NUM_TEC = 16
H_, W_, K_ = 56, 56, 49
RB = 8
NB = H_ // RB
HALO = RB + 2 * OFF
WK = W_ * K_
WKP = WK + 8
SLAB = K_ * RB * W_
PIXROW = W_ * NSP
NCHUNK = (WK + LANES - 1) // LANES


def _make_sc_body(bfix):
    def body(sims_hbm, sinds_hbm, tbl_hbm, out_hbm,
             sims_v, sinds_v, ph_v, tbl_v, mb_v):
        wid = lax.axis_index("c") * NUM_TEC + lax.axis_index("s")
        hb = wid // 4
        grp = wid - hb * 4
        r0 = hb * RB
        lo = jnp.clip(r0 - OFF, 0, H_ - HALO)
        s_lo = grp * 2
        s_hi = jnp.where(grp == 3, NSP, grp * 2 + 2)

        def work(_, __):
            inoff = pl.multiple_of((bfix * H_ + lo) * PIXROW, 8)
            pltpu.sync_copy(tbl_hbm, tbl_v)
            pltpu.sync_copy(sims_hbm.at[pl.ds(inoff, HALO * PIXROW)], sims_v)
            pltpu.sync_copy(sinds_hbm.at[pl.ds(inoff, HALO * PIXROW)],
                            sinds_v)

            zero = jnp.zeros((LANES,), jnp.float32)

            def zbody(i, _):
                ph_v[pl.ds(i * LANES, LANES)] = zero
                return 0
            lax.fori_loop(0, (HALO * WK) // LANES, zbody, 0)

            iota = lax.broadcasted_iota(jnp.int32, (LANES,), 0)
            pix9 = iota * NSP
            i49 = iota * K_

            def sbody(ci, _):
                for sp in range(NSP):
                    idxv = pix9 + (ci * (LANES * NSP) + sp)
                    sindv = plsc.load_gather(sinds_v, [idxv])
                    simsv = plsc.load_gather(sims_v, [idxv])
                    pidx = i49 + ci * (LANES * K_) + sindv
                    plsc.addupdate_scatter(ph_v, [pidx], simsv)
                return 0
            lax.fori_loop(0, (HALO * W_) // LANES, sbody, 0)

            def s_body(s, _):
                def cbody(c, _):
                    base = c * LANES
                    dhv = tbl_v[pl.ds(base, LANES)]
                    col49v = tbl_v[pl.ds(WKP + base, LANES)]
                    w9v = tbl_v[pl.ds(2 * WKP + base, LANES)]
                    didxv = tbl_v[pl.ds(3 * WKP + base, LANES)]
                    sidloc = w9v + s
                    for r in range(RB):
                        h = r0 + r
                        rv = jnp.clip(dhv + h, 0, H_ - 1)
                        geo = (rv - lo) * WK + col49v
                        sidv = plsc.load_gather(
                            sinds_v, [sidloc + (h - lo) * PIXROW])
                        val = plsc.load_gather(ph_v, [geo + sidv])
                        plsc.store_scatter(mb_v, [didxv + r * W_], val)
                    return 0
                lax.fori_loop(0, NCHUNK, cbody, 0)
                slaboff = (hb * NSP + s) * SLAB
                pltpu.sync_copy(mb_v.at[pl.ds(0, SLAB)],
                                out_hbm.at[pl.ds(pl.multiple_of(slaboff, 8),
                                                 SLAB)])
                return 0
            lax.fori_loop(s_lo, s_hi, s_body, 0)
            return 0
        lax.fori_loop(0, jnp.where(wid < 4 * NB, 1, 0), work, 0)
    return body


def _dense_body(c_ref, attn_ref, match_ref, out_ref):
    HD = attn_ref.shape[1]
    c = c_ref[0, 0]
    for hd in range(HD):
        e3 = jnp.exp(attn_ref[0, hd] - c)
        for s in range(NSP):
            m3 = match_ref[0, s]
            em3 = e3 * m3
            den = jnp.sum(em3, axis=0)
            out_ref[0, hd, s] = em3 * (1.0 / (EPS + den))[None]


def _dense_body2(c_ref, attn_ref, match_ref, prev_ref, out_ref):
    _dense_body(c_ref, attn_ref, match_ref, out_ref)


@jax.jit
def kernel(attn, sims, sinds):
    B, HD, H, W, K = attn.shape
    c = jnp.max(attn).reshape(1, 1)

    f = jnp.arange(WK, dtype=jnp.int32)
    wcol = f // K
    kk = f - wcol * K
    dh = kk // WS - OFF
    dw = kk - (kk // WS) * WS - OFF
    col49 = jnp.clip(wcol + dw, 0, W - 1) * K
    didx = kk * (RB * W) + wcol
    pad = jnp.zeros((WKP - WK,), jnp.int32)
    dpad = jnp.full((WKP - WK,), SLAB, jnp.int32)
    tbl = jnp.concatenate([dh, pad, col49, pad, wcol * NSP, pad,
                           didx, dpad]).astype(jnp.int32)

    sims_flat = sims.reshape(B * H * W * NSP)
    sinds_flat = sinds.reshape(B * H * W * NSP)

    mesh = plsc.VectorSubcoreMesh(core_axis_name="c", subcore_axis_name="s")
    scratch = [
        pltpu.VMEM((HALO * W * NSP,), jnp.float32),
        pltpu.VMEM((HALO * W * NSP,), jnp.int32),
        pltpu.VMEM((HALO * WK,), jnp.float32),
        pltpu.VMEM((4 * WKP,), jnp.int32),
        pltpu.VMEM((SLAB + RB * W,), jnp.float32),
    ]
    match_b = [
        pl.kernel(
            _make_sc_body(b),
            out_type=jax.ShapeDtypeStruct((NB * NSP * SLAB,), jnp.float32),
            mesh=mesh,
            compiler_params=pltpu.CompilerParams(needs_layout_passes=False),
            scratch_types=scratch,
        )(sims_flat, sinds_flat, tbl).reshape(NB, NSP, K, RB, W)
        for b in range(B)
    ]

    attn_t = jnp.transpose(attn, (0, 1, 4, 2, 3))
    out_shape = jax.ShapeDtypeStruct((B, HD, NSP, K, H, W), jnp.float32)

    def dense_call(b, body, extra_specs, aliases):
        return pl.pallas_call(
            body,
            grid=(NB,),
            in_specs=[
                pl.BlockSpec((1, 1), lambda hb: (0, 0)),
                pl.BlockSpec((1, HD, K, RB, W), lambda hb: (b, 0, 0, hb, 0)),
                pl.BlockSpec((1, NSP, K, RB, W), lambda hb: (hb, 0, 0, 0, 0)),
            ] + extra_specs,
            out_specs=pl.BlockSpec((1, HD, NSP, K, RB, W),
                                   lambda hb: (b, 0, 0, 0, hb, 0)),
            out_shape=out_shape,
            input_output_aliases=aliases,
            compiler_params=pltpu.CompilerParams(
                dimension_semantics=("parallel",)),
        )

    out0 = dense_call(0, _dense_body, [], {})(c, attn_t, match_b[0])
    out_phys = dense_call(
        1, _dense_body2, [pl.BlockSpec(memory_space=pl.ANY)], {3: 0},
    )(c, attn_t, match_b[1], out0)
    return jnp.transpose(out_phys, (0, 1, 2, 4, 5, 3))
